# baseline (speedup 1.0000x reference)
import functools

import jax
import jax.numpy as jnp
from jax import lax
from jax.experimental import pallas as pl
from jax.experimental.pallas import tpu as pltpu
from jax.experimental.pallas import tpu_sc as plsc

NC = 2
NS = 16
NW = NC * NS
CH = 128


def _chunk_assignment(t):
    base_n, extra = t // NW, t % NW

    def base_of(wid):
        return wid * base_n + jnp.minimum(wid, extra)

    def count_of(wid):
        return base_n + (wid < extra).astype(jnp.int32)

    return base_n, extra, base_of, count_of


def _sc_scalar_segsum(vals, rowp, colp, n, t, acc_rows, stripe,
                      count_only=False):
    zpad = ((stripe + 15) // 16) * 16
    base_n, extra, base_of, count_of = _chunk_assignment(t)
    kb = base_n + (1 if extra else 0)
    mesh = plsc.VectorSubcoreMesh(core_axis_name="c", subcore_axis_name="s")

    @functools.partial(
        pl.kernel,
        out_type=jax.ShapeDtypeStruct((NC * acc_rows,), jnp.float32),
        mesh=mesh,
        compiler_params=pltpu.CompilerParams(needs_layout_passes=False),
        scratch_types=[
            pltpu.VMEM((n,), jnp.float32),
            pltpu.VMEM((kb, 1, CH), jnp.int32),
            pltpu.VMEM((kb, 1, CH), jnp.int32),
            pltpu.VMEM((CH,), jnp.float32),
            pltpu.VMEM((zpad,), jnp.float32),
            pltpu.VMEM_SHARED((acc_rows,), jnp.float32),
        ],
    )
    def k_fn(vals_hbm, rowp_hbm, colp_hbm, out_hbm,
             vals_v, row_v, col_v, valbuf, zbuf, acc):
        cid = lax.axis_index("c")
        sid = lax.axis_index("s")
        wid = cid * NS + sid
        r0 = sid * stripe

        def zero_body(i, carry):
            zbuf[pl.ds(i * 16, 16)] = jnp.zeros((16,), jnp.float32)
            return carry
        lax.fori_loop(0, zpad // 16, zero_body, 0)
        pltpu.sync_copy(zbuf.at[pl.ds(0, stripe)], acc.at[pl.ds(r0, stripe)])
        plsc.subcore_barrier()

        base = base_of(wid)
        nc_ = count_of(wid)
        base_c = jnp.minimum(base, t - kb)
        off = base - base_c

        if not count_only:
            pltpu.sync_copy(vals_hbm, vals_v)
            pltpu.sync_copy(rowp_hbm.at[pl.ds(base_c, kb)], row_v)
        else:
            for v in range(CH // 16):
                valbuf[pl.ds(v * 16, 16)] = jnp.ones((16,), jnp.float32)
        pltpu.sync_copy(colp_hbm.at[pl.ds(base_c, kb)], col_v)

        def edge_body(j, carry):
            if not count_only:
                for v in range(CH // 16):
                    idx = row_v[off + j, 0, pl.ds(v * 16, 16)]
                    valbuf[pl.ds(v * 16, 16)] = plsc.load_gather(vals_v, [idx])
            pltpu.sync_copy(valbuf, acc.at[col_v.at[off + j, 0]], add=True)
            return carry
        lax.fori_loop(0, nc_, edge_body, 0)

        plsc.subcore_barrier()
        pltpu.sync_copy(acc.at[pl.ds(r0, stripe)], zbuf.at[pl.ds(0, stripe)])
        pltpu.sync_copy(zbuf.at[pl.ds(0, stripe)],
                        out_hbm.at[pl.ds(cid * acc_rows + r0, stripe)])

    return k_fn(vals, rowp, colp).reshape(NC, acc_rows)


def _sc_row_segsum(y, rowp, colp, t, acc_rows, stripe):
    n, f = y.shape
    base_n, extra, base_of, count_of = _chunk_assignment(t)
    npairs = base_n // 2
    mesh = plsc.VectorSubcoreMesh(core_axis_name="c", subcore_axis_name="s")

    @functools.partial(
        pl.kernel,
        out_type=jax.ShapeDtypeStruct((NC, acc_rows, f), jnp.float32),
        mesh=mesh,
        scratch_types=[
            pltpu.VMEM((1, CH), jnp.int32),
            pltpu.VMEM((1, CH), jnp.int32),
            pltpu.VMEM((1, CH), jnp.int32),
            pltpu.VMEM((1, CH), jnp.int32),
            pltpu.VMEM((CH, f), jnp.float32),
            pltpu.VMEM((CH, f), jnp.float32),
            pltpu.VMEM_SHARED((acc_rows, f), jnp.float32),
            pltpu.SemaphoreType.DMA,
            pltpu.SemaphoreType.DMA,
            pltpu.SemaphoreType.DMA,
            pltpu.SemaphoreType.DMA,
        ],
    )
    def k_fn(y_hbm, rowp_hbm, colp_hbm, out_hbm,
             ri_a, ri_b, ci_a, ci_b, buf_a, buf_b, acc,
             sem_a, sem_b, sem_ia, sem_ib):
        cid = lax.axis_index("c")
        sid = lax.axis_index("s")
        wid = cid * NS + sid
        r0 = sid * stripe
        base = base_of(wid)
        nc_ = count_of(wid)

        def idx_start(j, ri, ci, sem):
            pltpu.async_copy(rowp_hbm.at[base + j], ri, sem)
            pltpu.async_copy(colp_hbm.at[base + j], ci, sem)

        def idx_wait(j, ri, ci, sem):
            pltpu.make_async_copy(rowp_hbm.at[base + j], ri, sem).wait()
            pltpu.make_async_copy(colp_hbm.at[base + j], ci, sem).wait()

        def gather_start(ri, buf, sem):
            pltpu.async_copy(y_hbm.at[ri.at[0]], buf, sem)

        def gather_wait(ri, buf, sem):
            pltpu.make_async_copy(y_hbm.at[ri.at[0]], buf, sem).wait()

        idx_start(0, ri_a, ci_a, sem_ia)
        idx_start(1, ri_b, ci_b, sem_ib)

        def zr(r, carry):
            def zc(c, inner):
                buf_a[r, pl.ds(c * 16, 16)] = jnp.zeros((16,), jnp.float32)
                return inner
            return lax.fori_loop(0, f // 16, zc, carry)
        lax.fori_loop(0, CH, zr, 0)
        nfull, rem = stripe // CH, stripe % CH
        for q in range(nfull):
            pltpu.sync_copy(buf_a, acc.at[pl.ds(r0 + q * CH, CH)])
        if rem:
            pltpu.sync_copy(buf_a.at[pl.ds(0, rem)],
                            acc.at[pl.ds(r0 + nfull * CH, rem)])
        plsc.subcore_barrier()

        idx_wait(0, ri_a, ci_a, sem_ia)
        gather_start(ri_a, buf_a, sem_a)

        def body(i, carry):
            j0 = 2 * i
            gather_wait(ri_a, buf_a, sem_a)
            idx_wait(j0 + 1, ri_b, ci_b, sem_ib)
            gather_start(ri_b, buf_b, sem_b)
            pltpu.sync_copy(buf_a, acc.at[ci_a.at[0]], add=True)

            @pl.when(i < npairs - 1)
            def _():
                idx_start(j0 + 2, ri_a, ci_a, sem_ia)

            gather_wait(ri_b, buf_b, sem_b)

            @pl.when(i < npairs - 1)
            def _():
                idx_wait(j0 + 2, ri_a, ci_a, sem_ia)
                gather_start(ri_a, buf_a, sem_a)

            pltpu.sync_copy(buf_b, acc.at[ci_b.at[0]], add=True)

            @pl.when(i < npairs - 1)
            def _():
                idx_start(j0 + 3, ri_b, ci_b, sem_ib)
            return carry
        lax.fori_loop(0, npairs, body, 0)

        def tail_body(j, carry):
            idx_start(j, ri_a, ci_a, sem_ia)
            idx_wait(j, ri_a, ci_a, sem_ia)
            gather_start(ri_a, buf_a, sem_a)
            gather_wait(ri_a, buf_a, sem_a)
            pltpu.sync_copy(buf_a, acc.at[ci_a.at[0]], add=True)
            return carry
        lax.fori_loop(2 * npairs, nc_, tail_body, 0)

        plsc.subcore_barrier()
        pltpu.sync_copy(acc.at[pl.ds(r0, stripe), :],
                        out_hbm.at[cid, pl.ds(r0, stripe), :])

    return k_fn(y, rowp, colp)


def _tc_mm_heads(x, w1, o, wl, blr):
    n, f = x.shape
    br = 1000

    def body(x_ref, w_ref, o_ref, wl_ref, bl_ref, xw_ref, ohe_ref):
        xw_ref[...] = jnp.dot(x_ref[...], w_ref[...],
                              preferred_element_type=jnp.float32)
        ohe_ref[...] = jnp.dot(
            o_ref[...], wl_ref[...], preferred_element_type=jnp.float32
        ) + bl_ref[...]

    return pl.pallas_call(
        body,
        grid=(n // br,),
        in_specs=[
            pl.BlockSpec((br, f), lambda i: (i, 0)),
            pl.BlockSpec((f, f), lambda i: (0, 0)),
            pl.BlockSpec((br, 4), lambda i: (i, 0)),
            pl.BlockSpec((4, 1), lambda i: (0, 0)),
            pl.BlockSpec((1, 1), lambda i: (0, 0)),
        ],
        out_specs=[pl.BlockSpec((br, f), lambda i: (i, 0)),
                   pl.BlockSpec((br, 1), lambda i: (i, 0))],
        out_shape=[jax.ShapeDtypeStruct((n, f), jnp.float32),
                   jax.ShapeDtypeStruct((n, 1), jnp.float32)],
    )(x, w1, o, wl, blr)


def _tc_scale(xw, deg_t):
    n, f = xw.shape
    br = 1000

    def body(xw_ref, d_ref, y_ref, dinv_ref):
        deg = 1.0 + d_ref[:, 0:1] + d_ref[:, 1:2]
        dinv = lax.rsqrt(deg)
        y_ref[...] = dinv * xw_ref[...]
        dinv_ref[...] = dinv

    return pl.pallas_call(
        body,
        grid=(n // br,),
        in_specs=[
            pl.BlockSpec((br, f), lambda i: (i, 0)),
            pl.BlockSpec((br, NC), lambda i: (i, 0)),
        ],
        out_specs=[pl.BlockSpec((br, f), lambda i: (i, 0)),
                   pl.BlockSpec((br, 1), lambda i: (i, 0))],
        out_shape=[jax.ShapeDtypeStruct((n, f), jnp.float32),
                   jax.ShapeDtypeStruct((n, 1), jnp.float32)],
    )(xw, deg_t)


def _tc_layer2_in(sparts, y, dinv, b1r, w2):
    n, f = y.shape
    br = 1000

    def body(sp_ref, y_ref, d_ref, b1_ref, w2_ref, t_ref):
        s = sp_ref[0] + sp_ref[1]
        dinv = d_ref[...]
        h = jnp.maximum(dinv * (s + y_ref[...]) + b1_ref[...], 0.0)
        z = jnp.dot(h, w2_ref[...], preferred_element_type=jnp.float32)
        t_ref[...] = dinv * z

    return pl.pallas_call(
        body,
        grid=(n // br,),
        in_specs=[
            pl.BlockSpec((NC, br, f), lambda i: (0, i, 0)),
            pl.BlockSpec((br, f), lambda i: (i, 0)),
            pl.BlockSpec((br, 1), lambda i: (i, 0)),
            pl.BlockSpec((1, f), lambda i: (0, 0)),
            pl.BlockSpec((f, 1), lambda i: (0, 0)),
        ],
        out_specs=pl.BlockSpec((br, 1), lambda i: (i, 0)),
        out_shape=jax.ShapeDtypeStruct((n, 1), jnp.float32),
    )(sparts, y, dinv, b1r, w2)


def _tc_xm(s2_t, t, dinv, b2r):
    n = t.shape[0]
    br = 1000

    def body(s2_ref, t_ref, d_ref, b2_ref, xm_ref):
        s2 = s2_ref[:, 0:1] + s2_ref[:, 1:2]
        xm_ref[...] = d_ref[...] * (s2 + t_ref[...]) + b2_ref[...]

    return pl.pallas_call(
        body,
        grid=(n // br,),
        in_specs=[
            pl.BlockSpec((br, NC), lambda i: (i, 0)),
            pl.BlockSpec((br, 1), lambda i: (i, 0)),
            pl.BlockSpec((br, 1), lambda i: (i, 0)),
            pl.BlockSpec((1, 1), lambda i: (0, 0)),
        ],
        out_specs=pl.BlockSpec((br, 1), lambda i: (i, 0)),
        out_shape=jax.ShapeDtypeStruct((n, 1), jnp.float32),
    )(s2_t, t, dinv, b2r)


def _tc_broadcast(xm_row, ohe):
    n = ohe.shape[0]
    br = 80

    def body(xm_ref, ohe_ref, out_ref):
        out_ref[...] = ohe_ref[...] + xm_ref[...]

    return pl.pallas_call(
        body,
        grid=(n // br,),
        in_specs=[
            pl.BlockSpec((1, n), lambda i: (0, 0)),
            pl.BlockSpec((br, 1), lambda i: (i, 0)),
        ],
        out_specs=pl.BlockSpec((br, n), lambda i: (i, 0)),
        out_shape=jax.ShapeDtypeStruct((n, n), jnp.float32),
    )(xm_row, ohe)


def kernel(x, o, edge_index, W1, b1, W2, b2, Wl, bl):
    n, f = x.shape
    e = edge_index.shape[1]

    stripe = ((-(-n // NS) + 7) // 8) * 8
    acc_rows = stripe * NS
    if acc_rows <= n:
        stripe += 8
        acc_rows = stripe * NS

    ei = edge_index.astype(jnp.int32)
    if e % CH:
        padn = CH - e % CH
        pidx = jnp.arange(padn, dtype=jnp.int32)
        ei = jnp.concatenate(
            [ei, jnp.stack([pidx % n, n + pidx % (acc_rows - n)])], axis=1)
    t_chunks = ei.shape[1] // CH
    rowp = ei[0].reshape(t_chunks, 1, CH)
    colp = ei[1].reshape(t_chunks, 1, CH)

    xw, ohe = _tc_mm_heads(x, W1, o, Wl, bl.reshape(1, 1))
    ones = jnp.ones((n,), jnp.float32)
    degp = _sc_scalar_segsum(ones, rowp, colp, n, t_chunks, acc_rows, stripe,
                             count_only=True)
    deg_t = degp[:, :n].T

    y, dinv = _tc_scale(xw, deg_t)

    sparts = _sc_row_segsum(y, rowp, colp, t_chunks, acc_rows, stripe)

    t = _tc_layer2_in(sparts, y, dinv, b1.reshape(1, f), W2)

    s2p = _sc_scalar_segsum(t.reshape(n), rowp, colp, n, t_chunks,
                            acc_rows, stripe)
    s2_t = s2p[:, :n].T

    xm = _tc_xm(s2_t, t, dinv, b2.reshape(1, 1))
    return _tc_broadcast(xm.reshape(1, n), ohe)


# speedup vs baseline: 1.0528x; 1.0528x over previous
import functools

import jax
import jax.numpy as jnp
from jax import lax
from jax.experimental import pallas as pl
from jax.experimental.pallas import tpu as pltpu
from jax.experimental.pallas import tpu_sc as plsc

NC = 2
NS = 16
NW = NC * NS
CH = 128


def _chunk_assignment(t):
    base_n, extra = t // NW, t % NW

    def base_of(wid):
        return wid * base_n + jnp.minimum(wid, extra)

    def count_of(wid):
        return base_n + (wid < extra).astype(jnp.int32)

    return base_n, extra, base_of, count_of


def _sc_scalar_segsum(vals, rowp, colp, n, t, acc_rows, stripe,
                      count_only=False):
    zpad = ((stripe + 15) // 16) * 16
    base_n, extra, base_of, count_of = _chunk_assignment(t)
    kb = base_n + (1 if extra else 0)
    mesh = plsc.VectorSubcoreMesh(core_axis_name="c", subcore_axis_name="s")

    @functools.partial(
        pl.kernel,
        out_type=jax.ShapeDtypeStruct((NC * acc_rows,), jnp.float32),
        mesh=mesh,
        compiler_params=pltpu.CompilerParams(needs_layout_passes=False),
        scratch_types=[
            pltpu.VMEM((n,), jnp.float32),
            pltpu.VMEM((kb, 1, CH), jnp.int32),
            pltpu.VMEM((kb, 1, CH), jnp.int32),
            pltpu.VMEM((CH,), jnp.float32),
            pltpu.VMEM((zpad,), jnp.float32),
            pltpu.VMEM_SHARED((acc_rows,), jnp.float32),
        ],
    )
    def k_fn(vals_hbm, rowp_hbm, colp_hbm, out_hbm,
             vals_v, row_v, col_v, valbuf, zbuf, acc):
        cid = lax.axis_index("c")
        sid = lax.axis_index("s")
        wid = cid * NS + sid
        r0 = sid * stripe

        def zero_body(i, carry):
            zbuf[pl.ds(i * 16, 16)] = jnp.zeros((16,), jnp.float32)
            return carry
        lax.fori_loop(0, zpad // 16, zero_body, 0)
        pltpu.sync_copy(zbuf.at[pl.ds(0, stripe)], acc.at[pl.ds(r0, stripe)])
        plsc.subcore_barrier()

        base = base_of(wid)
        nc_ = count_of(wid)
        base_c = jnp.minimum(base, t - kb)
        off = base - base_c

        if not count_only:
            pltpu.sync_copy(vals_hbm, vals_v)
            pltpu.sync_copy(rowp_hbm.at[pl.ds(base_c, kb)], row_v)
        else:
            for v in range(CH // 16):
                valbuf[pl.ds(v * 16, 16)] = jnp.ones((16,), jnp.float32)
        pltpu.sync_copy(colp_hbm.at[pl.ds(base_c, kb)], col_v)

        def edge_body(j, carry):
            if not count_only:
                for v in range(CH // 16):
                    idx = row_v[off + j, 0, pl.ds(v * 16, 16)]
                    valbuf[pl.ds(v * 16, 16)] = plsc.load_gather(vals_v, [idx])
            pltpu.sync_copy(valbuf, acc.at[col_v.at[off + j, 0]], add=True)
            return carry
        lax.fori_loop(0, nc_, edge_body, 0)

        plsc.subcore_barrier()
        pltpu.sync_copy(acc.at[pl.ds(r0, stripe)], zbuf.at[pl.ds(0, stripe)])
        pltpu.sync_copy(zbuf.at[pl.ds(0, stripe)],
                        out_hbm.at[pl.ds(cid * acc_rows + r0, stripe)])

    return k_fn(vals, rowp, colp).reshape(NC, acc_rows)


def _sc_row_segsum(y, rowp, colp, t, acc_rows, stripe):
    n, f = y.shape
    base_n, extra, base_of, count_of = _chunk_assignment(t)
    npairs = base_n // 2
    mesh = plsc.VectorSubcoreMesh(core_axis_name="c", subcore_axis_name="s")

    @functools.partial(
        pl.kernel,
        out_type=jax.ShapeDtypeStruct((NC, acc_rows, f), jnp.float32),
        mesh=mesh,
        scratch_types=[
            pltpu.VMEM((1, CH), jnp.int32),
            pltpu.VMEM((1, CH), jnp.int32),
            pltpu.VMEM((1, CH), jnp.int32),
            pltpu.VMEM((1, CH), jnp.int32),
            pltpu.VMEM((CH, f), jnp.float32),
            pltpu.VMEM((CH, f), jnp.float32),
            pltpu.VMEM_SHARED((acc_rows, f), jnp.float32),
            pltpu.SemaphoreType.DMA,
            pltpu.SemaphoreType.DMA,
            pltpu.SemaphoreType.DMA,
            pltpu.SemaphoreType.DMA,
        ],
    )
    def k_fn(y_hbm, rowp_hbm, colp_hbm, out_hbm,
             ri_a, ri_b, ci_a, ci_b, buf_a, buf_b, acc,
             sem_a, sem_b, sem_ia, sem_ib):
        cid = lax.axis_index("c")
        sid = lax.axis_index("s")
        wid = cid * NS + sid
        r0 = sid * stripe
        base = base_of(wid)
        nc_ = count_of(wid)

        def idx_start(j, ri, ci, sem):
            pltpu.async_copy(rowp_hbm.at[base + j], ri, sem)
            pltpu.async_copy(colp_hbm.at[base + j], ci, sem)

        def idx_wait(j, ri, ci, sem):
            pltpu.make_async_copy(rowp_hbm.at[base + j], ri, sem).wait()
            pltpu.make_async_copy(colp_hbm.at[base + j], ci, sem).wait()

        def gather_start(ri, buf, sem):
            pltpu.async_copy(y_hbm.at[ri.at[0]], buf, sem)

        def gather_wait(ri, buf, sem):
            pltpu.make_async_copy(y_hbm.at[ri.at[0]], buf, sem).wait()

        idx_start(0, ri_a, ci_a, sem_ia)
        idx_start(1, ri_b, ci_b, sem_ib)

        def zr(r, carry):
            def zc(c, inner):
                buf_a[r, pl.ds(c * 16, 16)] = jnp.zeros((16,), jnp.float32)
                return inner
            return lax.fori_loop(0, f // 16, zc, carry)
        lax.fori_loop(0, CH, zr, 0)
        nfull, rem = stripe // CH, stripe % CH
        for q in range(nfull):
            pltpu.sync_copy(buf_a, acc.at[pl.ds(r0 + q * CH, CH)])
        if rem:
            pltpu.sync_copy(buf_a.at[pl.ds(0, rem)],
                            acc.at[pl.ds(r0 + nfull * CH, rem)])
        plsc.subcore_barrier()

        idx_wait(0, ri_a, ci_a, sem_ia)
        gather_start(ri_a, buf_a, sem_a)

        def body(i, carry):
            j0 = 2 * i
            gather_wait(ri_a, buf_a, sem_a)
            idx_wait(j0 + 1, ri_b, ci_b, sem_ib)
            gather_start(ri_b, buf_b, sem_b)
            pltpu.sync_copy(buf_a, acc.at[ci_a.at[0]], add=True)

            @pl.when(i < npairs - 1)
            def _():
                idx_start(j0 + 2, ri_a, ci_a, sem_ia)

            gather_wait(ri_b, buf_b, sem_b)

            @pl.when(i < npairs - 1)
            def _():
                idx_wait(j0 + 2, ri_a, ci_a, sem_ia)
                gather_start(ri_a, buf_a, sem_a)

            pltpu.sync_copy(buf_b, acc.at[ci_b.at[0]], add=True)

            @pl.when(i < npairs - 1)
            def _():
                idx_start(j0 + 3, ri_b, ci_b, sem_ib)
            return carry
        lax.fori_loop(0, npairs, body, 0)

        def tail_body(j, carry):
            idx_start(j, ri_a, ci_a, sem_ia)
            idx_wait(j, ri_a, ci_a, sem_ia)
            gather_start(ri_a, buf_a, sem_a)
            gather_wait(ri_a, buf_a, sem_a)
            pltpu.sync_copy(buf_a, acc.at[ci_a.at[0]], add=True)
            return carry
        lax.fori_loop(2 * npairs, nc_, tail_body, 0)

        plsc.subcore_barrier()
        pltpu.sync_copy(acc.at[pl.ds(r0, stripe), :],
                        out_hbm.at[cid, pl.ds(r0, stripe), :])

    return k_fn(y, rowp, colp)


def _tc_mm_heads(x, w1, o, wl, blr):
    n, f = x.shape
    br = 1000

    def body(x_ref, w_ref, o_ref, wl_ref, bl_ref, xw_ref, ohe_ref):
        xw_ref[...] = jnp.dot(x_ref[...], w_ref[...],
                              preferred_element_type=jnp.float32)
        ohe_ref[...] = jnp.dot(
            o_ref[...], wl_ref[...], preferred_element_type=jnp.float32
        ) + bl_ref[...]

    return pl.pallas_call(
        body,
        grid=(n // br,),
        in_specs=[
            pl.BlockSpec((br, f), lambda i: (i, 0)),
            pl.BlockSpec((f, f), lambda i: (0, 0)),
            pl.BlockSpec((br, 4), lambda i: (i, 0)),
            pl.BlockSpec((4, 1), lambda i: (0, 0)),
            pl.BlockSpec((1, 1), lambda i: (0, 0)),
        ],
        out_specs=[pl.BlockSpec((br, f), lambda i: (i, 0)),
                   pl.BlockSpec((br, 1), lambda i: (i, 0))],
        out_shape=[jax.ShapeDtypeStruct((n, f), jnp.float32),
                   jax.ShapeDtypeStruct((n, 1), jnp.float32)],
    )(x, w1, o, wl, blr)


def _tc_scale(xw, deg_t):
    n, f = xw.shape
    br = 1000

    def body(xw_ref, d_ref, y_ref, dinv_ref):
        deg = 1.0 + d_ref[:, 0:1] + d_ref[:, 1:2]
        dinv = lax.rsqrt(deg)
        y_ref[...] = dinv * xw_ref[...]
        dinv_ref[...] = dinv

    return pl.pallas_call(
        body,
        grid=(n // br,),
        in_specs=[
            pl.BlockSpec((br, f), lambda i: (i, 0)),
            pl.BlockSpec((br, NC), lambda i: (i, 0)),
        ],
        out_specs=[pl.BlockSpec((br, f), lambda i: (i, 0)),
                   pl.BlockSpec((br, 1), lambda i: (i, 0))],
        out_shape=[jax.ShapeDtypeStruct((n, f), jnp.float32),
                   jax.ShapeDtypeStruct((n, 1), jnp.float32)],
    )(xw, deg_t)


def _tc_layer2_in(sparts, y, dinv, b1r, w2):
    n, f = y.shape
    br = 1000

    def body(sp_ref, y_ref, d_ref, b1_ref, w2_ref, t_ref):
        s = sp_ref[0] + sp_ref[1]
        dinv = d_ref[...]
        h = jnp.maximum(dinv * (s + y_ref[...]) + b1_ref[...], 0.0)
        z = jnp.dot(h, w2_ref[...], preferred_element_type=jnp.float32)
        t_ref[...] = dinv * z

    return pl.pallas_call(
        body,
        grid=(n // br,),
        in_specs=[
            pl.BlockSpec((NC, br, f), lambda i: (0, i, 0)),
            pl.BlockSpec((br, f), lambda i: (i, 0)),
            pl.BlockSpec((br, 1), lambda i: (i, 0)),
            pl.BlockSpec((1, f), lambda i: (0, 0)),
            pl.BlockSpec((f, 1), lambda i: (0, 0)),
        ],
        out_specs=pl.BlockSpec((br, 1), lambda i: (i, 0)),
        out_shape=jax.ShapeDtypeStruct((n, 1), jnp.float32),
    )(sparts, y, dinv, b1r, w2)


def _tc_xm(s2_t, t, dinv, b2r):
    n = t.shape[0]
    br = 1000

    def body(s2_ref, t_ref, d_ref, b2_ref, xm_ref):
        s2 = s2_ref[:, 0:1] + s2_ref[:, 1:2]
        xm_ref[...] = d_ref[...] * (s2 + t_ref[...]) + b2_ref[...]

    return pl.pallas_call(
        body,
        grid=(n // br,),
        in_specs=[
            pl.BlockSpec((br, NC), lambda i: (i, 0)),
            pl.BlockSpec((br, 1), lambda i: (i, 0)),
            pl.BlockSpec((br, 1), lambda i: (i, 0)),
            pl.BlockSpec((1, 1), lambda i: (0, 0)),
        ],
        out_specs=pl.BlockSpec((br, 1), lambda i: (i, 0)),
        out_shape=jax.ShapeDtypeStruct((n, 1), jnp.float32),
    )(s2_t, t, dinv, b2r)


def _tc_broadcast(xm_row, ohe):
    n = ohe.shape[0]
    br = 200

    def body(xm_ref, ohe_ref, out_ref):
        out_ref[...] = ohe_ref[...] + xm_ref[...]

    return pl.pallas_call(
        body,
        grid=(n // br,),
        in_specs=[
            pl.BlockSpec((1, n), lambda i: (0, 0)),
            pl.BlockSpec((br, 1), lambda i: (i, 0)),
        ],
        out_specs=pl.BlockSpec((br, n), lambda i: (i, 0)),
        out_shape=jax.ShapeDtypeStruct((n, n), jnp.float32),
    )(xm_row, ohe)


def kernel(x, o, edge_index, W1, b1, W2, b2, Wl, bl):
    n, f = x.shape
    e = edge_index.shape[1]

    stripe = ((-(-n // NS) + 7) // 8) * 8
    acc_rows = stripe * NS
    if acc_rows <= n:
        stripe += 8
        acc_rows = stripe * NS

    ei = edge_index.astype(jnp.int32)
    if e % CH:
        padn = CH - e % CH
        pidx = jnp.arange(padn, dtype=jnp.int32)
        ei = jnp.concatenate(
            [ei, jnp.stack([pidx % n, n + pidx % (acc_rows - n)])], axis=1)
    t_chunks = ei.shape[1] // CH
    rowp = ei[0].reshape(t_chunks, 1, CH)
    colp = ei[1].reshape(t_chunks, 1, CH)

    xw, ohe = _tc_mm_heads(x, W1, o, Wl, bl.reshape(1, 1))
    ones = jnp.ones((n,), jnp.float32)
    degp = _sc_scalar_segsum(ones, rowp, colp, n, t_chunks, acc_rows, stripe,
                             count_only=True)
    deg_t = degp[:, :n].T

    y, dinv = _tc_scale(xw, deg_t)

    sparts = _sc_row_segsum(y, rowp, colp, t_chunks, acc_rows, stripe)

    t = _tc_layer2_in(sparts, y, dinv, b1.reshape(1, f), W2)

    s2p = _sc_scalar_segsum(t.reshape(n), rowp, colp, n, t_chunks,
                            acc_rows, stripe)
    s2_t = s2p[:, :n].T

    xm = _tc_xm(s2_t, t, dinv, b2.reshape(1, 1))
    return _tc_broadcast(xm.reshape(1, n), ohe)
